# baseline (speedup 1.0000x reference)
import contextlib

import jax
import jax.numpy as jnp
from jax import lax
from jax.experimental import pallas as pl
from jax.experimental.pallas import tpu as pltpu
from jax.experimental.pallas import tpu_sc as plsc

_M = 1048576
_D = 64
_B = 524288

_NCORE = 2
_NSUB = 16
_BT = _B // _NSUB
_NCHUNK = 128
_CPC = _NCHUNK // _NCORE
_CR = _M // _NCHUNK
_CHUNK_SHIFT = 13
_DUMMY = _CR
_G = 64
_SETS = 4
_RPT = _CR // _NSUB
_FB = 128
_NFB = _RPT // _FB
_PIECE = 512
_ZR = 32
_NBC = _CPC * _NFB


def _named_range(name, n):
    with jax.named_scope(name):
        yield from range(n)


def _sc_body(mem_hbm, counts_hbm, val_hbm, idx_hbm, out_hbm, ncnt_hbm,
             idx_piece, poslist, hist, binstart, binoff,
             gi0, gi1, gi2, gi3, si0, si1, si2, si3,
             rw0, rw1, rw2, rw3, ones, zeros2d, zeros1d,
             mb0, mb1, ob0, ob1, acc_buf, cb0, cb1, cacc_buf, nc_buf,
             acc, cnt_acc,
             sg0, sg1, sg2, sg3, sa0, sa1, sa2, sa3,
             sm0, sm1, sc0, sc1, so0, so1, sem_zero,
             sem_accr, sem_caccr, sem_nc):
    sem_gath = (sg0, sg1, sg2, sg3)
    sem_add = (sa0, sa1, sa2, sa3)
    sem_memin = (sm0, sm1)
    sem_cntin = (sc0, sc1)
    sem_out = (so0, so1)
    gis = (gi0, gi1, gi2, gi3)
    sis = (si0, si1, si2, si3)
    rws = (rw0, rw1, rw2, rw3)
    mbs = (mb0, mb1)
    obs = (ob0, ob1)
    cbs = (cb0, cb1)

    c = lax.axis_index("c")
    s = lax.axis_index("s")
    tb = s * _BT
    iota = lax.iota(jnp.int32, 16)
    z16 = jnp.zeros((16,), jnp.float32)

    def _fill_zeros2d(r, _):
        for u in range(_D // 16):
            zeros2d[r, pl.ds(u * 16, 16)] = z16
        return 0
    lax.fori_loop(0, _ZR, _fill_zeros2d, 0)

    def _fill_zeros1d(i, _):
        zeros1d[pl.ds(i * 16, 16)] = z16
        return 0
    lax.fori_loop(0, _RPT // 16, _fill_zeros1d, 0)

    for u in range(_G // 16):
        ones[pl.ds(u * 16, 16)] = jnp.ones((16,), jnp.float32)

    r0l = s * _RPT
    for z in range(_RPT // _ZR):
        pltpu.make_async_copy(
            zeros2d, acc.at[pl.ds(r0l + z * _ZR, _ZR)], sem_zero).start()
    pltpu.make_async_copy(
        zeros1d, cnt_acc.at[pl.ds(r0l, _RPT)], sem_zero).start()
    for p in range(2):
        ra0 = c * _CPC * _CR + r0l + p * _FB
        pltpu.make_async_copy(
            mem_hbm.at[pl.ds(ra0, _FB)], mbs[p], sem_memin[p]).start()
        pltpu.make_async_copy(
            counts_hbm.at[pl.ds(ra0, _FB)], cbs[p], sem_cntin[p]).start()

    rbase, _ = plsc.scan_count(jnp.zeros((16,), jnp.int32))
    basis = rbase[0]

    for v in range(_NCHUNK // 16 + 1):
        hist[pl.ds(v * 16, 16)] = jnp.zeros((16,), jnp.int32)

    def _hist_piece(pc, _):
        pltpu.sync_copy(idx_hbm.at[pl.ds(tb + pc * _PIECE, _PIECE)], idx_piece)

        def _hist_body(j, _):
            v = idx_piece[pl.ds(j * 16, 16)]
            b = lax.shift_right_logical(v, _CHUNK_SHIFT)
            bs, _ps = plsc.sort_key_val(b, iota)
            rank, last = plsc.scan_count(bs)
            plsc.addupdate_scatter(hist, [bs], rank - basis + 1, mask=last)
            return 0
        lax.fori_loop(0, _PIECE // 16, _hist_body, 0)
        return 0
    with jax.named_scope("bin_hist"):
        lax.fori_loop(0, _BT // _PIECE, _hist_piece, 0)

    total = jnp.int32(0)
    for v in range(_NCHUNK // 16):
        hv = hist[pl.ds(v * 16, 16)]
        cs = plsc.cumsum(hv) + total
        ex = cs - hv
        binstart[pl.ds(v * 16, 16)] = ex
        binoff[pl.ds(v * 16, 16)] = ex
        total = cs[15]
    binstart[pl.ds(_NCHUNK, 16)] = jnp.full((16,), _BT, jnp.int32)
    binoff[pl.ds(_NCHUNK, 16)] = jnp.full((16,), _BT, jnp.int32)

    def _place_piece(pc, _):
        pltpu.sync_copy(idx_hbm.at[pl.ds(tb + pc * _PIECE, _PIECE)], idx_piece)

        def _place_body(j, _):
            v = idx_piece[pl.ds(j * 16, 16)]
            b = lax.shift_right_logical(v, _CHUNK_SHIFT)
            lr = v & (_CR - 1)
            pos = pc * _PIECE + j * 16 + iota
            packed = lax.shift_left(lr, 16) | pos
            bs, pk = plsc.sort_key_val(b, packed)
            rank, last = plsc.scan_count(bs)
            r0 = rank - basis
            base = plsc.load_gather(binoff, [bs])
            plsc.store_scatter(poslist, [base + r0], pk)
            plsc.store_scatter(binoff, [bs], base + r0 + 1, mask=last)
            return 0
        lax.fori_loop(0, _PIECE // 16, _place_body, 0)
        return 0
    with jax.named_scope("bin_place"):
        lax.fori_loop(0, _BT // _PIECE, _place_piece, 0)

    def _drain_zeros():
        for z in range(_RPT // _ZR):
            pltpu.make_async_copy(
                zeros2d, acc.at[pl.ds(r0l, _ZR)], sem_zero).wait()
        pltpu.make_async_copy(
            zeros1d, cnt_acc.at[pl.ds(r0l, _RPT)], sem_zero).wait()

    def _chunk_body(chl, _):
        g = c * _CPC + chl
        with jax.named_scope("sc_sync"):
            _drain_zeros()
            plsc.subcore_barrier()

        se = plsc.load_gather(binstart, [g + jnp.minimum(iota, 1)])
        start = se[0]
        end = se[1]
        nblk = lax.div(end - start + _G - 1, _G)

        def _stage(n, i):
            for u in range(_G // 16):
                kvec = start + n * _G + u * 16 + iota
                valid = kvec < end
                kc = jnp.minimum(kvec, end - 1)
                pk = plsc.load_gather(poslist, [kc])
                pos = pk & 0xFFFF
                lr = lax.shift_right_logical(pk, 16)
                gis[i][pl.ds(u * 16, 16)] = tb + pos
                sis[i][pl.ds(u * 16, 16)] = jnp.where(valid, lr, _DUMMY)

        def _souter(bb, _):
            n0 = bb * _SETS
            for i in range(_SETS):
                n = n0 + i

                @pl.when(n < nblk)
                def _():
                    @pl.when(n >= _SETS)
                    def _():
                        pltpu.make_async_copy(
                            rws[i], acc.at[sis[i]], sem_add[i]).wait()
                        pltpu.make_async_copy(
                            ones, cnt_acc.at[sis[i]], sem_add[i]).wait()
                    _stage(n, i)
                    pltpu.make_async_copy(
                        val_hbm.at[gis[i]], rws[i], sem_gath[i]).start()
            for i in range(_SETS):
                n = n0 + i

                @pl.when(n < nblk)
                def _():
                    pltpu.make_async_copy(
                        val_hbm.at[gis[i]], rws[i], sem_gath[i]).wait()
                    pltpu.make_async_copy(
                        rws[i], acc.at[sis[i]], sem_add[i]).start(add=True)
                    pltpu.make_async_copy(
                        ones, cnt_acc.at[sis[i]], sem_add[i]).start(add=True)
            return 0
        with jax.named_scope("sc_scatter"):
            lax.fori_loop(0, lax.div(nblk + _SETS - 1, _SETS), _souter, 0)

        for i in range(_SETS):
            @pl.when(i < nblk)
            def _():
                pltpu.make_async_copy(rws[i], acc.at[sis[i]], sem_add[i]).wait()
                pltpu.make_async_copy(ones, cnt_acc.at[sis[i]], sem_add[i]).wait()
        plsc.subcore_barrier()

        pltpu.make_async_copy(
            acc.at[pl.ds(r0l, _FB)], acc_buf, sem_accr).start()
        pltpu.make_async_copy(
            cnt_acc.at[pl.ds(r0l, _RPT)], cacc_buf, sem_caccr).start()
        for k in _named_range("sc_final", _NFB):
            bc = chl * _NFB + k
            p = k % 2
            ra = g * _CR + r0l + k * _FB
            la = r0l + k * _FB
            pltpu.make_async_copy(
                mem_hbm.at[pl.ds(ra, _FB)], mbs[p], sem_memin[p]).wait()
            if k == 0:
                pltpu.make_async_copy(
                    cnt_acc.at[pl.ds(r0l, _RPT)], cacc_buf, sem_caccr).wait()
                pltpu.make_async_copy(
                    zeros1d, cnt_acc.at[pl.ds(r0l, _RPT)], sem_zero).start()

                @pl.when(chl >= 1)
                def _():
                    pltpu.make_async_copy(
                        nc_buf, ncnt_hbm.at[pl.ds(g * _CR + r0l, _RPT)],
                        sem_nc).wait()
            pltpu.make_async_copy(
                acc.at[pl.ds(la, _FB)], acc_buf, sem_accr).wait()
            for z in range(_FB // _ZR):
                pltpu.make_async_copy(
                    zeros2d, acc.at[pl.ds(la + z * _ZR, _ZR)],
                    sem_zero).start()
            pltpu.make_async_copy(
                counts_hbm.at[pl.ds(ra, _FB)], cbs[p], sem_cntin[p]).wait()

            @pl.when(bc >= 2)
            def _():
                pltpu.make_async_copy(
                    obs[p], out_hbm.at[pl.ds(ra, _FB)], sem_out[p]).wait()

            def _rowvec_body(rv, _):
                cv = cbs[p][pl.ds(rv * 16, 16)]
                av = cacc_buf[pl.ds(k * _FB + rv * 16, 16)]
                ncv = cv + av
                inv = 1.0 / jnp.maximum(ncv, 1.0)
                nc_buf[pl.ds(k * _FB + rv * 16, 16)] = ncv
                mw = cv * inv
                for u in range(16):
                    r = rv * 16 + u
                    a = mw[u]
                    w = inv[u]
                    for q in range(_D // 16):
                        m0 = mbs[p][r, pl.ds(q * 16, 16)]
                        a0 = acc_buf[r, pl.ds(q * 16, 16)]
                        obs[p][r, pl.ds(q * 16, 16)] = m0 * a + a0 * w
                return 0
            lax.fori_loop(0, _FB // 16, _rowvec_body, 0)

            if k < _NFB - 1:
                pltpu.make_async_copy(
                    acc.at[pl.ds(la + _FB, _FB)], acc_buf, sem_accr).start()
            pltpu.make_async_copy(
                obs[p], out_hbm.at[pl.ds(ra, _FB)], sem_out[p]).start()
            if k == _NFB - 1:
                pltpu.make_async_copy(
                    nc_buf, ncnt_hbm.at[pl.ds(g * _CR + r0l, _RPT)],
                    sem_nc).start()

            nbc = bc + 2

            @pl.when(nbc < _NBC)
            def _():
                g2 = c * _CPC + lax.div(nbc, _NFB)
                ra2 = g2 * _CR + r0l + lax.rem(nbc, _NFB) * _FB
                pltpu.make_async_copy(
                    mem_hbm.at[pl.ds(ra2, _FB)], mbs[p], sem_memin[p]).start()
                pltpu.make_async_copy(
                    counts_hbm.at[pl.ds(ra2, _FB)], cbs[p], sem_cntin[p]).start()
        return 0
    lax.fori_loop(0, _CPC, _chunk_body, 0)

    for p in range(2):
        pltpu.make_async_copy(
            obs[p], out_hbm.at[pl.ds(r0l, _FB)], sem_out[p]).wait()
    pltpu.make_async_copy(
        nc_buf, ncnt_hbm.at[pl.ds(r0l, _RPT)], sem_nc).wait()
    _drain_zeros()


@jax.jit
def _sc_pipeline(mem, counts, val, idx):
    mesh = plsc.VectorSubcoreMesh(core_axis_name="c", subcore_axis_name="s",
                                  num_cores=_NCORE, num_subcores=_NSUB)
    f = pl.kernel(
        _sc_body,
        out_type=[
            jax.ShapeDtypeStruct((_M, _D), jnp.float32),
            jax.ShapeDtypeStruct((_M,), jnp.float32),
        ],
        mesh=mesh,
        compiler_params=pltpu.CompilerParams(
            needs_layout_passes=False, use_tc_tiling_on_sc=False),
        scratch_types=[
            pltpu.VMEM((_PIECE,), jnp.int32),
            pltpu.VMEM((_BT,), jnp.int32),
            pltpu.VMEM((_NCHUNK + 16,), jnp.int32),
            pltpu.VMEM((_NCHUNK + 16,), jnp.int32),
            pltpu.VMEM((_NCHUNK + 16,), jnp.int32),
        ] + [pltpu.VMEM((_G,), jnp.int32)] * 8
        + [pltpu.VMEM((_G, _D), jnp.float32)] * 4
        + [
            pltpu.VMEM((_G,), jnp.float32),
            pltpu.VMEM((_ZR, _D), jnp.float32),
            pltpu.VMEM((_RPT,), jnp.float32),
            pltpu.VMEM((_FB, _D), jnp.float32),
            pltpu.VMEM((_FB, _D), jnp.float32),
            pltpu.VMEM((_FB, _D), jnp.float32),
            pltpu.VMEM((_FB, _D), jnp.float32),
            pltpu.VMEM((_FB, _D), jnp.float32),
            pltpu.VMEM((_FB,), jnp.float32),
            pltpu.VMEM((_FB,), jnp.float32),
            pltpu.VMEM((_RPT,), jnp.float32),
            pltpu.VMEM((_RPT,), jnp.float32),
            pltpu.VMEM_SHARED((_CR + 8, _D), jnp.float32),
            pltpu.VMEM_SHARED((_CR + 8,), jnp.float32),
        ] + [pltpu.SemaphoreType.DMA] * 18,
    )
    return f(mem, counts, val, idx)


def kernel(mem, counts, val, idx):
    new_mem, new_counts = _sc_pipeline(mem, counts, val, idx)
    return new_mem, new_counts


# speedup vs baseline: 1.0171x; 1.0171x over previous
import contextlib

import jax
import jax.numpy as jnp
from jax import lax
from jax.experimental import pallas as pl
from jax.experimental.pallas import tpu as pltpu
from jax.experimental.pallas import tpu_sc as plsc

_M = 1048576
_D = 64
_B = 524288

_NCORE = 2
_NSUB = 16
_BT = _B // _NSUB
_NCHUNK = 128
_CPC = _NCHUNK // _NCORE
_CR = _M // _NCHUNK
_CHUNK_SHIFT = 13
_DUMMY = _CR
_G = 64
_SETS = 4
_RPT = _CR // _NSUB
_FB = 128
_NFB = _RPT // _FB
_PIECE = 2048
_ZR = 32
_NBC = _CPC * _NFB


def _named_range(name, n):
    with jax.named_scope(name):
        yield from range(n)


def _sc_body(mem_hbm, counts_hbm, val_hbm, idx_hbm, out_hbm, ncnt_hbm,
             idx_piece, poslist, hist, binstart, binoff,
             gi0, gi1, gi2, gi3, si0, si1, si2, si3,
             rw0, rw1, rw2, rw3, ones, zeros2d, zeros1d,
             mb0, mb1, ob0, ob1, acc_buf, cb0, cb1, cacc_buf, nc_buf,
             acc, cnt_acc,
             sg0, sg1, sg2, sg3, sa0, sa1, sa2, sa3,
             sm0, sm1, sc0, sc1, so0, so1, sem_zero,
             sem_accr, sem_caccr, sem_nc):
    sem_gath = (sg0, sg1, sg2, sg3)
    sem_add = (sa0, sa1, sa2, sa3)
    sem_memin = (sm0, sm1)
    sem_cntin = (sc0, sc1)
    sem_out = (so0, so1)
    gis = (gi0, gi1, gi2, gi3)
    sis = (si0, si1, si2, si3)
    rws = (rw0, rw1, rw2, rw3)
    mbs = (mb0, mb1)
    obs = (ob0, ob1)
    cbs = (cb0, cb1)

    c = lax.axis_index("c")
    s = lax.axis_index("s")
    tb = s * _BT
    iota = lax.iota(jnp.int32, 16)
    z16 = jnp.zeros((16,), jnp.float32)

    def _fill_zeros2d(r, _):
        for u in range(_D // 16):
            zeros2d[r, pl.ds(u * 16, 16)] = z16
        return 0
    lax.fori_loop(0, _ZR, _fill_zeros2d, 0)

    def _fill_zeros1d(i, _):
        zeros1d[pl.ds(i * 16, 16)] = z16
        return 0
    lax.fori_loop(0, _RPT // 16, _fill_zeros1d, 0)

    for u in range(_G // 16):
        ones[pl.ds(u * 16, 16)] = jnp.ones((16,), jnp.float32)

    r0l = s * _RPT
    for z in range(_RPT // _ZR):
        pltpu.make_async_copy(
            zeros2d, acc.at[pl.ds(r0l + z * _ZR, _ZR)], sem_zero).start()
    pltpu.make_async_copy(
        zeros1d, cnt_acc.at[pl.ds(r0l, _RPT)], sem_zero).start()
    for p in range(2):
        ra0 = c * _CPC * _CR + r0l + p * _FB
        pltpu.make_async_copy(
            mem_hbm.at[pl.ds(ra0, _FB)], mbs[p], sem_memin[p]).start()
        pltpu.make_async_copy(
            counts_hbm.at[pl.ds(ra0, _FB)], cbs[p], sem_cntin[p]).start()

    rbase, _ = plsc.scan_count(jnp.zeros((16,), jnp.int32))
    basis = rbase[0]

    for v in range(_NCHUNK // 16 + 1):
        hist[pl.ds(v * 16, 16)] = jnp.zeros((16,), jnp.int32)

    def _hist_piece(pc, _):
        pltpu.sync_copy(idx_hbm.at[pl.ds(tb + pc * _PIECE, _PIECE)], idx_piece)

        def _hist_body(j, _):
            v = idx_piece[pl.ds(j * 16, 16)]
            b = lax.shift_right_logical(v, _CHUNK_SHIFT)
            bs, _ps = plsc.sort_key_val(b, iota)
            rank, last = plsc.scan_count(bs)
            plsc.addupdate_scatter(hist, [bs], rank - basis + 1, mask=last)
            return 0
        lax.fori_loop(0, _PIECE // 16, _hist_body, 0)
        return 0
    with jax.named_scope("bin_hist"):
        lax.fori_loop(0, _BT // _PIECE, _hist_piece, 0)

    total = jnp.int32(0)
    for v in range(_NCHUNK // 16):
        hv = hist[pl.ds(v * 16, 16)]
        cs = plsc.cumsum(hv) + total
        ex = cs - hv
        binstart[pl.ds(v * 16, 16)] = ex
        binoff[pl.ds(v * 16, 16)] = ex
        total = cs[15]
    binstart[pl.ds(_NCHUNK, 16)] = jnp.full((16,), _BT, jnp.int32)
    binoff[pl.ds(_NCHUNK, 16)] = jnp.full((16,), _BT, jnp.int32)

    def _place_piece(pc, _):
        pltpu.sync_copy(idx_hbm.at[pl.ds(tb + pc * _PIECE, _PIECE)], idx_piece)

        def _place_body(j, _):
            v = idx_piece[pl.ds(j * 16, 16)]
            b = lax.shift_right_logical(v, _CHUNK_SHIFT)
            lr = v & (_CR - 1)
            pos = pc * _PIECE + j * 16 + iota
            packed = lax.shift_left(lr, 16) | pos
            bs, pk = plsc.sort_key_val(b, packed)
            rank, last = plsc.scan_count(bs)
            r0 = rank - basis
            base = plsc.load_gather(binoff, [bs])
            plsc.store_scatter(poslist, [base + r0], pk)
            plsc.store_scatter(binoff, [bs], base + r0 + 1, mask=last)
            return 0
        lax.fori_loop(0, _PIECE // 16, _place_body, 0)
        return 0
    with jax.named_scope("bin_place"):
        lax.fori_loop(0, _BT // _PIECE, _place_piece, 0)

    def _drain_zeros():
        for z in range(_RPT // _ZR):
            pltpu.make_async_copy(
                zeros2d, acc.at[pl.ds(r0l, _ZR)], sem_zero).wait()
        pltpu.make_async_copy(
            zeros1d, cnt_acc.at[pl.ds(r0l, _RPT)], sem_zero).wait()

    def _chunk_body(chl, _):
        g = c * _CPC + chl
        with jax.named_scope("sc_sync"):
            _drain_zeros()
            plsc.subcore_barrier()

        se = plsc.load_gather(binstart, [g + jnp.minimum(iota, 1)])
        start = se[0]
        end = se[1]
        nblk = lax.div(end - start + _G - 1, _G)

        def _stage(n, i):
            for u in range(_G // 16):
                kvec = start + n * _G + u * 16 + iota
                valid = kvec < end
                kc = jnp.minimum(kvec, end - 1)
                pk = plsc.load_gather(poslist, [kc])
                pos = pk & 0xFFFF
                lr = lax.shift_right_logical(pk, 16)
                gis[i][pl.ds(u * 16, 16)] = tb + pos
                sis[i][pl.ds(u * 16, 16)] = jnp.where(valid, lr, _DUMMY)

        def _souter(bb, _):
            n0 = bb * _SETS
            for i in range(_SETS):
                n = n0 + i

                @pl.when(n < nblk)
                def _():
                    @pl.when(n >= _SETS)
                    def _():
                        pltpu.make_async_copy(
                            rws[i], acc.at[sis[i]], sem_add[i]).wait()
                        pltpu.make_async_copy(
                            ones, cnt_acc.at[sis[i]], sem_add[i]).wait()
                    _stage(n, i)
                    pltpu.make_async_copy(
                        val_hbm.at[gis[i]], rws[i], sem_gath[i]).start()
            for i in range(_SETS):
                n = n0 + i

                @pl.when(n < nblk)
                def _():
                    pltpu.make_async_copy(
                        val_hbm.at[gis[i]], rws[i], sem_gath[i]).wait()
                    pltpu.make_async_copy(
                        rws[i], acc.at[sis[i]], sem_add[i]).start(add=True)
                    pltpu.make_async_copy(
                        ones, cnt_acc.at[sis[i]], sem_add[i]).start(add=True)
            return 0
        with jax.named_scope("sc_scatter"):
            lax.fori_loop(0, lax.div(nblk + _SETS - 1, _SETS), _souter, 0)

        for i in range(_SETS):
            @pl.when(i < nblk)
            def _():
                pltpu.make_async_copy(rws[i], acc.at[sis[i]], sem_add[i]).wait()
                pltpu.make_async_copy(ones, cnt_acc.at[sis[i]], sem_add[i]).wait()
        plsc.subcore_barrier()

        pltpu.make_async_copy(
            acc.at[pl.ds(r0l, _FB)], acc_buf, sem_accr).start()
        pltpu.make_async_copy(
            cnt_acc.at[pl.ds(r0l, _RPT)], cacc_buf, sem_caccr).start()
        for k in _named_range("sc_final", _NFB):
            bc = chl * _NFB + k
            p = k % 2
            ra = g * _CR + r0l + k * _FB
            la = r0l + k * _FB
            pltpu.make_async_copy(
                mem_hbm.at[pl.ds(ra, _FB)], mbs[p], sem_memin[p]).wait()
            if k == 0:
                pltpu.make_async_copy(
                    cnt_acc.at[pl.ds(r0l, _RPT)], cacc_buf, sem_caccr).wait()
                pltpu.make_async_copy(
                    zeros1d, cnt_acc.at[pl.ds(r0l, _RPT)], sem_zero).start()

                @pl.when(chl >= 1)
                def _():
                    pltpu.make_async_copy(
                        nc_buf, ncnt_hbm.at[pl.ds(g * _CR + r0l, _RPT)],
                        sem_nc).wait()
            pltpu.make_async_copy(
                acc.at[pl.ds(la, _FB)], acc_buf, sem_accr).wait()
            for z in range(_FB // _ZR):
                pltpu.make_async_copy(
                    zeros2d, acc.at[pl.ds(la + z * _ZR, _ZR)],
                    sem_zero).start()
            pltpu.make_async_copy(
                counts_hbm.at[pl.ds(ra, _FB)], cbs[p], sem_cntin[p]).wait()

            @pl.when(bc >= 2)
            def _():
                pltpu.make_async_copy(
                    obs[p], out_hbm.at[pl.ds(ra, _FB)], sem_out[p]).wait()

            def _rowvec_body(rv, _):
                cv = cbs[p][pl.ds(rv * 16, 16)]
                av = cacc_buf[pl.ds(k * _FB + rv * 16, 16)]
                ncv = cv + av
                inv = 1.0 / jnp.maximum(ncv, 1.0)
                nc_buf[pl.ds(k * _FB + rv * 16, 16)] = ncv
                mw = cv * inv
                for u in range(16):
                    r = rv * 16 + u
                    a = mw[u]
                    w = inv[u]
                    for q in range(_D // 16):
                        m0 = mbs[p][r, pl.ds(q * 16, 16)]
                        a0 = acc_buf[r, pl.ds(q * 16, 16)]
                        obs[p][r, pl.ds(q * 16, 16)] = m0 * a + a0 * w
                return 0
            lax.fori_loop(0, _FB // 16, _rowvec_body, 0)

            if k < _NFB - 1:
                pltpu.make_async_copy(
                    acc.at[pl.ds(la + _FB, _FB)], acc_buf, sem_accr).start()
            pltpu.make_async_copy(
                obs[p], out_hbm.at[pl.ds(ra, _FB)], sem_out[p]).start()
            if k == _NFB - 1:
                pltpu.make_async_copy(
                    nc_buf, ncnt_hbm.at[pl.ds(g * _CR + r0l, _RPT)],
                    sem_nc).start()

            nbc = bc + 2

            @pl.when(nbc < _NBC)
            def _():
                g2 = c * _CPC + lax.div(nbc, _NFB)
                ra2 = g2 * _CR + r0l + lax.rem(nbc, _NFB) * _FB
                pltpu.make_async_copy(
                    mem_hbm.at[pl.ds(ra2, _FB)], mbs[p], sem_memin[p]).start()
                pltpu.make_async_copy(
                    counts_hbm.at[pl.ds(ra2, _FB)], cbs[p], sem_cntin[p]).start()
        return 0
    lax.fori_loop(0, _CPC, _chunk_body, 0)

    for p in range(2):
        pltpu.make_async_copy(
            obs[p], out_hbm.at[pl.ds(r0l, _FB)], sem_out[p]).wait()
    pltpu.make_async_copy(
        nc_buf, ncnt_hbm.at[pl.ds(r0l, _RPT)], sem_nc).wait()
    _drain_zeros()


@jax.jit
def _sc_pipeline(mem, counts, val, idx):
    mesh = plsc.VectorSubcoreMesh(core_axis_name="c", subcore_axis_name="s",
                                  num_cores=_NCORE, num_subcores=_NSUB)
    f = pl.kernel(
        _sc_body,
        out_type=[
            jax.ShapeDtypeStruct((_M, _D), jnp.float32),
            jax.ShapeDtypeStruct((_M,), jnp.float32),
        ],
        mesh=mesh,
        compiler_params=pltpu.CompilerParams(
            needs_layout_passes=False, use_tc_tiling_on_sc=False),
        scratch_types=[
            pltpu.VMEM((_PIECE,), jnp.int32),
            pltpu.VMEM((_BT,), jnp.int32),
            pltpu.VMEM((_NCHUNK + 16,), jnp.int32),
            pltpu.VMEM((_NCHUNK + 16,), jnp.int32),
            pltpu.VMEM((_NCHUNK + 16,), jnp.int32),
        ] + [pltpu.VMEM((_G,), jnp.int32)] * 8
        + [pltpu.VMEM((_G, _D), jnp.float32)] * 4
        + [
            pltpu.VMEM((_G,), jnp.float32),
            pltpu.VMEM((_ZR, _D), jnp.float32),
            pltpu.VMEM((_RPT,), jnp.float32),
            pltpu.VMEM((_FB, _D), jnp.float32),
            pltpu.VMEM((_FB, _D), jnp.float32),
            pltpu.VMEM((_FB, _D), jnp.float32),
            pltpu.VMEM((_FB, _D), jnp.float32),
            pltpu.VMEM((_FB, _D), jnp.float32),
            pltpu.VMEM((_FB,), jnp.float32),
            pltpu.VMEM((_FB,), jnp.float32),
            pltpu.VMEM((_RPT,), jnp.float32),
            pltpu.VMEM((_RPT,), jnp.float32),
            pltpu.VMEM_SHARED((_CR + 8, _D), jnp.float32),
            pltpu.VMEM_SHARED((_CR + 8,), jnp.float32),
        ] + [pltpu.SemaphoreType.DMA] * 18,
    )
    return f(mem, counts, val, idx)


def kernel(mem, counts, val, idx):
    new_mem, new_counts = _sc_pipeline(mem, counts, val, idx)
    return new_mem, new_counts
